# baseline (speedup 1.0000x reference)
import math

import jax
import jax.numpy as jnp
from jax.experimental import pallas as pl

_TI = 32


def _fast_sincos(x):
    two_over_pi = 0.6366197723675814
    p1, p2 = 1.5703125, 4.837512969970703125e-4
    k = jnp.floor(x * two_over_pi + 0.5)
    r = (x - k * p1) - k * p2
    z = r * r
    sp = r * (1.0 + z * (-0.16663813 + z * 0.00817275))
    cp = 1.0 + z * (-0.49980093 + z * 0.04054557)
    ki = k.astype(jnp.int32)
    swap = (ki & 1) == 1
    s1 = jnp.where(swap, cp, sp)
    c1 = jnp.where(swap, sp, cp)
    s = jnp.where((ki & 2) == 2, -s1, s1)
    c = jnp.where(((ki + 1) & 2) == 2, -c1, c1)
    return s, c


def _egnn_body(XiP_ref, XjP_ref, f4_ref, ft_ref, na_ref,
               Win_ref, bin_ref, Win4_ref, bin4_ref, Whi_ref, Whj_ref,
               Whj4_ref, be1_ref,
               Rep12_ref, Wsc_ref, Wes_ref, Wec_ref,
               We2_ref, be2_ref, Wx1_ref, bx1_ref, Wx2_ref, bx2_ref,
               Wh1h_ref, Wh1a_ref, Wh1n_ref, bh1_ref, Wh2_ref, bh2_ref,
               Wout_ref, bout_ref, ex_ref, ef_ref):
    ti = ft_ref.shape[1]
    n4 = f4_ref.shape[1]
    n = 4 * n4
    h = Whi_ref.shape[0]
    silu = jax.nn.silu

    fi = ft_ref[0]
    f4 = f4_ref[0]
    h4 = f4 @ Win4_ref[...] + bin4_ref[...]
    Bp4 = h4 @ Whj4_ref[...]
    h_i = fi @ Win_ref[...] + bin_ref[...]
    A = h_i @ Whi_ref[...]
    Bd = h_i @ Whj_ref[...]

    pd = XiP_ref[0] - XjP_ref[0]
    sq = pd * pd
    r4 = sq[:, 0:4] + sq[:, 4:8] + sq[:, 8:12]
    d4 = jnp.sqrt(r4 + 1e-8)
    inv4 = 1.0 / (d4 + 1.0)

    d4h = d4.astype(jnp.bfloat16).astype(jnp.float32)
    d4l = d4 - d4h
    lhs12 = jnp.concatenate([d4h, d4h, d4l], axis=1)
    ang4 = lhs12 @ Rep12_ref[...]
    s4, c4 = _fast_sincos(ang4)
    SC = jnp.concatenate([s4, c4, r4], axis=1)
    Ee = SC @ Wsc_ref[...]
    Ab = A + be1_ref[...]
    A4 = jnp.concatenate([Ab, Ab, Ab, Ab], axis=1)
    pre = Ee.reshape(ti, n4, 4 * h) + A4[:, None, :] + Bp4[None, :, :]
    m1 = silu(pre).reshape(ti, n4, 4, h).reshape(ti * n, h)
    m = silu(m1 @ We2_ref[...] + be2_ref[...])
    p1 = silu(m @ Wx1_ref[...] + bx1_ref[...])
    phi = p1 @ Wx2_ref[...] + bx2_ref[...]

    w4 = inv4 * phi.reshape(ti * n4, 4)
    scale = 1.0 / (n - 1)
    tx = jnp.sum(jnp.sum((pd[:, 0:4] * w4).reshape(ti, n4, 4), axis=1),
                 axis=1, keepdims=True)
    ty = jnp.sum(jnp.sum((pd[:, 4:8] * w4).reshape(ti, n4, 4), axis=1),
                 axis=1, keepdims=True)
    tz = jnp.sum(jnp.sum((pd[:, 8:12] * w4).reshape(ti, n4, 4), axis=1),
                 axis=1, keepdims=True)
    ex_ref[0] = jnp.concatenate([tx, ty, tz], axis=1) * scale

    aggf = jnp.sum(m.reshape(ti, n, h), axis=1)
    dd = math.sqrt(1e-8)
    cc = -math.log(10000.0) / 32.0
    frr = jnp.exp(jax.lax.broadcasted_iota(jnp.int32, (1, 32), 1)
                  .astype(jnp.float32) * cc)
    sd, cd = _fast_sincos(dd * frr)
    pred = Ab + Bd + sd @ Wes_ref[...] + cd @ Wec_ref[...]
    md = silu(silu(pred) @ We2_ref[...] + be2_ref[...])
    agg = aggf - md

    u = silu(h_i @ Wh1h_ref[...] + agg @ Wh1a_ref[...]
             + na_ref[...] @ Wh1n_ref[...] + bh1_ref[...])
    hn = h_i + u @ Wh2_ref[...] + bh2_ref[...]
    hout = hn @ Wout_ref[...] + bout_ref[...]
    ef_ref[0] = hout - fi


def _sin_embed(vals, dim):
    half = dim // 2
    freqs = jnp.exp(jnp.arange(half, dtype=jnp.float32)
                    * (-math.log(10000.0) / half))
    ang = vals[..., None] * freqs
    return jnp.concatenate([jnp.sin(ang), jnp.cos(ang)], axis=-1)


def kernel(coordinates, features, idx, W_in, b_in, We1, be1, We2, be2,
           Wx1, bx1, Wx2, bx2, Wh1, bh1, Wh2, bh2, W_out, b_out):
    x = coordinates.astype(jnp.float32)
    f = features.astype(jnp.float32)
    b, n, _ = x.shape
    nf = f.shape[-1]
    h = W_in.shape[-1]
    ti = _TI
    n4 = n // 4

    pos = _sin_embed(jnp.arange(n, dtype=jnp.float32), h)
    temb = _sin_embed(jnp.full((1,), idx, dtype=jnp.float32), h)
    na = (pos + temb).astype(jnp.float32)

    Whi, Whj = We1[:h], We1[h:2 * h]
    wr = We1[2 * h]
    Wes = We1[2 * h + 1:2 * h + 33]
    Wec = We1[2 * h + 33:2 * h + 65]
    Wh1h, Wh1a, Wh1n = Wh1[:h], Wh1[h:2 * h], Wh1[2 * h:]

    half = 32
    freqs = jnp.exp(jnp.arange(half, dtype=jnp.float32)
                    * (-math.log(10000.0) / half))
    lane = jnp.arange(4 * half)
    slot = lane // half
    RepF = jnp.where(slot[None, :] == jnp.arange(4)[:, None],
                     jnp.tile(freqs, 4)[None, :], 0.0)
    RepFh = RepF.astype(jnp.bfloat16).astype(jnp.float32)
    RepFl = RepF - RepFh
    Rep12 = jnp.concatenate([RepFh, RepFl, RepFh], axis=0)
    eye4 = jnp.eye(4, dtype=jnp.float32)
    RepW = (jnp.kron(eye4, wr.reshape(1, h)))
    Ws4 = jnp.kron(eye4, Wes)
    Wc4 = jnp.kron(eye4, Wec)
    Wsc = jnp.concatenate([Ws4, Wc4, RepW], axis=0)
    Win4 = jnp.kron(eye4, W_in)
    Whj4 = jnp.kron(eye4, Whj)
    bin4 = jnp.tile(b_in.reshape(1, -1), (1, 4))
    r1 = lambda v: v.reshape(1, -1)

    x12 = jnp.repeat(x, 4, axis=2)
    XiP = jnp.repeat(x12, n4, axis=1)
    xj12 = jnp.transpose(x.reshape(b, n4, 4, 3),
                         (0, 1, 3, 2)).reshape(b, n4, 12)
    XjP = jnp.tile(xj12, (1, ti, 1))
    f4 = f.reshape(b, n4, 4 * nf)

    grid = (b, n // ti)
    full = lambda s: pl.BlockSpec(s, lambda bi, t: (0,) * len(s))
    in_specs = [
        pl.BlockSpec((1, ti * n4, 12), lambda bi, t: (bi, t, 0)),
        pl.BlockSpec((1, ti * n4, 12), lambda bi, t: (bi, 0, 0)),
        pl.BlockSpec((1, n4, 4 * nf), lambda bi, t: (bi, 0, 0)),
        pl.BlockSpec((1, ti, nf), lambda bi, t: (bi, t, 0)),
        pl.BlockSpec((ti, h), lambda bi, t: (t, 0)),
        full(W_in.shape), full((1, h)), full(Win4.shape), full((1, 4 * h)),
        full(Whi.shape), full(Whj.shape), full(Whj4.shape),
        full((1, h)), full(Rep12.shape), full(Wsc.shape),
        full(Wes.shape), full(Wec.shape),
        full(We2.shape), full((1, h)), full(Wx1.shape), full((1, h)),
        full(Wx2.shape), full((1, 1)), full(Wh1h.shape), full(Wh1a.shape),
        full(Wh1n.shape), full((1, h)), full(Wh2.shape), full((1, h)),
        full(W_out.shape), full((1, nf)),
    ]
    out_specs = [
        pl.BlockSpec((1, ti, 3), lambda bi, t: (bi, t, 0)),
        pl.BlockSpec((1, ti, nf), lambda bi, t: (bi, t, 0)),
    ]
    ex, ef = pl.pallas_call(
        _egnn_body,
        grid=grid,
        in_specs=in_specs,
        out_specs=out_specs,
        out_shape=[
            jax.ShapeDtypeStruct((b, n, 3), jnp.float32),
            jax.ShapeDtypeStruct((b, n, nf), jnp.float32),
        ],
    )(XiP, XjP, f4, f, na,
      W_in, r1(b_in), Win4, bin4, Whi, Whj, Whj4, r1(be1),
      Rep12, Wsc, Wes, Wec, We2, r1(be2),
      Wx1, r1(bx1), Wx2, bx2.reshape(1, 1), Wh1h, Wh1a, Wh1n, r1(bh1),
      Wh2, r1(bh2), W_out, r1(b_out))
    return (ex, ef)


# speedup vs baseline: 1.1103x; 1.1103x over previous
import math

import jax
import jax.numpy as jnp
from jax.experimental import pallas as pl

_TI = 32


def _fast_sincos(x):
    two_over_pi = 0.6366197723675814
    p1, p2 = 1.5703125, 4.837512969970703125e-4
    k = jnp.floor(x * two_over_pi + 0.5)
    r = (x - k * p1) - k * p2
    z = r * r
    sp = r * (1.0 + z * (-0.16663813 + z * 0.00817275))
    cp = 1.0 + z * (-0.49980093 + z * 0.04054557)
    ki = k.astype(jnp.int32)
    swap = (ki & 1) == 1
    s1 = jnp.where(swap, cp, sp)
    c1 = jnp.where(swap, sp, cp)
    s = jnp.where((ki & 2) == 2, -s1, s1)
    c = jnp.where(((ki + 1) & 2) == 2, -c1, c1)
    return s, c


def _egnn_body(x_ref, xT_ref, ff_ref, ft_ref, na_ref,
               Win_ref, bin_ref, Whi_ref, Whj_ref, W66_ref,
               We2_ref, be2_ref, Wx1_ref, bx1_ref, Wx2_ref, bx2_ref,
               Wh1h_ref, Wh1a_ref, Wh1n_ref, bh1_ref, Wh2_ref, bh2_ref,
               Wout_ref, bout_ref, ex_ref, ef_ref):
    ti = x_ref.shape[1]
    n = xT_ref.shape[2]
    silu = jax.nn.silu

    fj = ff_ref[0]
    fi = ft_ref[0]
    Win = Win_ref[...]
    bin_ = bin_ref[...]
    h_j = fj @ Win + bin_
    h_i = fi @ Win + bin_
    Bp = h_j @ Whj_ref[...]
    A = h_i @ Whi_ref[...]
    Bd = h_i @ Whj_ref[...]

    xi = x_ref[0]
    xT = xT_ref[0]
    dx = xi[:, 0:1] - xT[0:1, :]
    dy = xi[:, 1:2] - xT[1:2, :]
    dz = xi[:, 2:3] - xT[2:3, :]
    radial = dx * dx + dy * dy + dz * dz
    dist = jnp.sqrt(radial + 1e-8)
    inv = 1.0 / (dist + 1.0)

    c = -math.log(10000.0) / 32.0
    fr = jnp.exp(jax.lax.broadcasted_iota(jnp.int32, (1, 1, 32), 2)
                 .astype(jnp.float32) * c)
    ang = dist[:, :, None] * fr
    es, ec = _fast_sincos(ang)
    S = jnp.concatenate(
        [es, ec, radial[:, :, None], jnp.ones((ti, n, 1), jnp.float32)],
        axis=-1).reshape(ti * n, 66)
    Ee = S @ W66_ref[...]

    pre = Ee.reshape(ti, n, -1) + A[:, None, :] + Bp[None, :, :]
    m1 = silu(pre).reshape(ti * n, -1)
    m = silu(m1 @ We2_ref[...] + be2_ref[...])
    p1 = silu(m @ Wx1_ref[...] + bx1_ref[...])
    phi = p1 @ Wx2_ref[...] + bx2_ref[...]

    w = inv * phi.reshape(ti, n)
    tx = jnp.sum(dx * w, axis=1, keepdims=True)
    ty = jnp.sum(dy * w, axis=1, keepdims=True)
    tz = jnp.sum(dz * w, axis=1, keepdims=True)
    ex_ref[0] = jnp.concatenate([tx, ty, tz], axis=1) * (1.0 / (n - 1))

    aggf = jnp.sum(m.reshape(ti, n, -1), axis=1)
    dd = math.sqrt(1e-8)
    frr = fr.reshape(1, 32)
    sd, cd = _fast_sincos(dd * frr)
    Sd = jnp.concatenate(
        [sd, cd, jnp.zeros((1, 1), jnp.float32),
         jnp.ones((1, 1), jnp.float32)], axis=1)
    pred = A + Bd + Sd @ W66_ref[...]
    md = silu(silu(pred) @ We2_ref[...] + be2_ref[...])
    agg = aggf - md

    u = silu(h_i @ Wh1h_ref[...] + agg @ Wh1a_ref[...]
             + na_ref[...] @ Wh1n_ref[...] + bh1_ref[...])
    hn = h_i + u @ Wh2_ref[...] + bh2_ref[...]
    hout = hn @ Wout_ref[...] + bout_ref[...]
    ef_ref[0] = hout - fi


def _sin_embed(vals, dim):
    half = dim // 2
    freqs = jnp.exp(jnp.arange(half, dtype=jnp.float32)
                    * (-math.log(10000.0) / half))
    ang = vals[..., None] * freqs
    return jnp.concatenate([jnp.sin(ang), jnp.cos(ang)], axis=-1)


def kernel(coordinates, features, idx, W_in, b_in, We1, be1, We2, be2,
           Wx1, bx1, Wx2, bx2, Wh1, bh1, Wh2, bh2, W_out, b_out):
    x = coordinates.astype(jnp.float32)
    f = features.astype(jnp.float32)
    b, n, _ = x.shape
    nf = f.shape[-1]
    h = W_in.shape[-1]
    ti = _TI

    xT = jnp.swapaxes(x, 1, 2)

    pos = _sin_embed(jnp.arange(n, dtype=jnp.float32), h)
    temb = _sin_embed(jnp.full((1,), idx, dtype=jnp.float32), h)
    na = (pos + temb).astype(jnp.float32)

    Whi, Whj = We1[:h], We1[h:2 * h]
    W66 = jnp.concatenate(
        [We1[2 * h + 1:2 * h + 33], We1[2 * h + 33:2 * h + 65],
         We1[2 * h:2 * h + 1], be1.reshape(1, -1)], axis=0)
    Wh1h, Wh1a, Wh1n = Wh1[:h], Wh1[h:2 * h], Wh1[2 * h:]
    r1 = lambda v: v.reshape(1, -1)

    grid = (b, n // ti)
    full = lambda s: pl.BlockSpec(s, lambda bi, t: (0,) * len(s))
    in_specs = [
        pl.BlockSpec((1, ti, 3), lambda bi, t: (bi, t, 0)),
        pl.BlockSpec((1, 3, n), lambda bi, t: (bi, 0, 0)),
        pl.BlockSpec((1, n, nf), lambda bi, t: (bi, 0, 0)),
        pl.BlockSpec((1, ti, nf), lambda bi, t: (bi, t, 0)),
        pl.BlockSpec((ti, h), lambda bi, t: (t, 0)),
        full(W_in.shape), full((1, h)), full(Whi.shape), full(Whj.shape),
        full(W66.shape),
        full(We2.shape), full((1, h)), full(Wx1.shape), full((1, h)),
        full(Wx2.shape), full((1, 1)), full(Wh1h.shape), full(Wh1a.shape),
        full(Wh1n.shape), full((1, h)), full(Wh2.shape), full((1, h)),
        full(W_out.shape), full((1, nf)),
    ]
    out_specs = [
        pl.BlockSpec((1, ti, 3), lambda bi, t: (bi, t, 0)),
        pl.BlockSpec((1, ti, nf), lambda bi, t: (bi, t, 0)),
    ]
    ex, ef = pl.pallas_call(
        _egnn_body,
        grid=grid,
        in_specs=in_specs,
        out_specs=out_specs,
        out_shape=[
            jax.ShapeDtypeStruct((b, n, 3), jnp.float32),
            jax.ShapeDtypeStruct((b, n, nf), jnp.float32),
        ],
    )(x, xT, f, f, na,
      W_in, r1(b_in), Whi, Whj, W66, We2, r1(be2),
      Wx1, r1(bx1), Wx2, bx2.reshape(1, 1), Wh1h, Wh1a, Wh1n, r1(bh1),
      Wh2, r1(bh2), W_out, r1(b_out))
    return (ex, ef)
